# baseline (speedup 1.0000x reference)
import jax
import jax.numpy as jnp
from jax import lax
from jax.experimental import pallas as pl
from jax.experimental.pallas import tpu as pltpu
from jax.experimental.pallas import tpu_sc as plsc

_JITTER_EPS = 0.1
_TOK_BLK = 512

_SC_CORES = 2
_SC_SUBCORES = 16
_SC_LANES = 16
_SC_WORKERS = _SC_CORES * _SC_SUBCORES


def _tc_body(lg_ref, om_ref, mult_ref):
    lg = lg_ref[...]
    mx = jnp.max(lg, axis=-1, keepdims=True)
    factor = jnp.maximum(jnp.abs(lg), mx)
    mask = (mx - lg) / factor > 2.0 * _JITTER_EPS
    e = jnp.where(mask, 0.0, jnp.exp(lg - mx))
    m = 1.0 / jnp.sum(e, axis=-1, keepdims=True)
    mult_ref[...] = m * om_ref[...][None, :]


def _tc_multiplier(logits, omega):
    n_tok, n_exp = logits.shape
    dim = omega.shape[0]
    return pl.pallas_call(
        _tc_body,
        grid=(n_tok // _TOK_BLK,),
        in_specs=[
            pl.BlockSpec((_TOK_BLK, n_exp), lambda i: (i, 0)),
            pl.BlockSpec((dim,), lambda i: (0,)),
        ],
        out_specs=pl.BlockSpec((_TOK_BLK, dim), lambda i: (i, 0)),
        out_shape=jax.ShapeDtypeStruct((n_tok, dim), jnp.float32),
        compiler_params=pltpu.CompilerParams(vmem_limit_bytes=40 * 1024 * 1024),
    )(logits, omega)


def _sc_sample(logits):
    n_tok, n_exp = logits.shape
    tpw = n_tok // _SC_WORKERS
    n_grp = tpw // _SC_LANES

    def body(lg_hbm, out_hbm, lg_v, samp_v):
        wid = lax.axis_index("s") * _SC_CORES + lax.axis_index("c")
        base = wid * tpw
        pltpu.sync_copy(lg_hbm.at[pl.ds(base, tpw), :], lg_v)

        def group(g, carry):
            rows = g * _SC_LANES + lax.iota(jnp.int32, _SC_LANES)
            best_v = plsc.load_gather(lg_v, [rows, jnp.zeros((_SC_LANES,), jnp.int32)])
            best_i = jnp.zeros((_SC_LANES,), jnp.int32)
            for j in range(1, n_exp):
                v = plsc.load_gather(lg_v, [rows, jnp.full((_SC_LANES,), j, jnp.int32)])
                upd = v > best_v
                best_v = jnp.where(upd, v, best_v)
                best_i = jnp.where(upd, jnp.full((_SC_LANES,), j, jnp.int32), best_i)
            samp_v[pl.ds(g * _SC_LANES, _SC_LANES)] = best_i
            return carry

        lax.fori_loop(0, n_grp, group, 0)
        pltpu.sync_copy(samp_v, out_hbm.at[pl.ds(base, tpw)])

    return pl.kernel(
        body,
        mesh=plsc.VectorSubcoreMesh(core_axis_name="c", subcore_axis_name="s"),
        compiler_params=pltpu.CompilerParams(
            needs_layout_passes=False, vmem_limit_bytes=8 * 1024 * 1024
        ),
        out_type=jax.ShapeDtypeStruct((n_tok,), jnp.int32),
        scratch_types=[
            pltpu.VMEM((tpw, n_exp), jnp.float32),
            pltpu.VMEM((tpw,), jnp.int32),
        ],
    )(logits)


def kernel(logits, omega):
    multiplier = _tc_multiplier(logits, omega)
    sample = _sc_sample(logits)
    return sample.reshape(-1, 1), multiplier, jnp.float32(0.0)


# speedup vs baseline: 1.0566x; 1.0566x over previous
import jax
import jax.numpy as jnp
from jax import lax
from jax.experimental import pallas as pl
from jax.experimental.pallas import tpu as pltpu
from jax.experimental.pallas import tpu_sc as plsc

_JITTER_EPS = 0.1
_TOK_BLK = 512

_SC_CORES = 2
_SC_SUBCORES = 16
_SC_LANES = 16
_SC_WORKERS = _SC_CORES * _SC_SUBCORES


def _tc_body(lg_ref, om_ref, mult_ref):
    lg = lg_ref[...]
    mx = jnp.max(lg, axis=0, keepdims=True)
    factor = jnp.maximum(jnp.abs(lg), mx)
    mask = (mx - lg) / factor > 2.0 * _JITTER_EPS
    e = jnp.where(mask, 0.0, jnp.exp(lg - mx))
    m = 1.0 / jnp.sum(e, axis=0, keepdims=True)
    mult_ref[...] = m.reshape(-1, 1) * om_ref[...][None, :]


def _tc_multiplier(logits_t, omega):
    n_exp, n_tok = logits_t.shape
    dim = omega.shape[0]
    return pl.pallas_call(
        _tc_body,
        grid=(n_tok // _TOK_BLK,),
        in_specs=[
            pl.BlockSpec((n_exp, _TOK_BLK), lambda i: (0, i)),
            pl.BlockSpec((dim,), lambda i: (0,)),
        ],
        out_specs=pl.BlockSpec((_TOK_BLK, dim), lambda i: (i, 0)),
        out_shape=jax.ShapeDtypeStruct((n_tok, dim), jnp.float32),
        compiler_params=pltpu.CompilerParams(vmem_limit_bytes=40 * 1024 * 1024),
    )(logits_t, omega)


def _sc_sample(logits_t):
    n_exp, n_tok = logits_t.shape
    tpw = n_tok // _SC_WORKERS
    n_grp = tpw // _SC_LANES

    def body(lg_hbm, out_hbm, lg_v, samp_v):
        wid = lax.axis_index("s") * _SC_CORES + lax.axis_index("c")
        base = wid * tpw
        pltpu.sync_copy(lg_hbm.at[:, pl.ds(base, tpw)], lg_v)

        def group(g, carry):
            off = g * _SC_LANES
            best_v = lg_v[0, pl.ds(off, _SC_LANES)]
            best_i = jnp.zeros((_SC_LANES,), jnp.int32)
            for j in range(1, n_exp):
                v = lg_v[j, pl.ds(off, _SC_LANES)]
                upd = v > best_v
                best_v = jnp.where(upd, v, best_v)
                best_i = jnp.where(upd, jnp.full((_SC_LANES,), j, jnp.int32), best_i)
            samp_v[pl.ds(off, _SC_LANES)] = best_i
            return carry

        lax.fori_loop(0, n_grp, group, 0)
        pltpu.sync_copy(samp_v, out_hbm.at[pl.ds(base, tpw)])

    return pl.kernel(
        body,
        mesh=plsc.VectorSubcoreMesh(core_axis_name="c", subcore_axis_name="s"),
        compiler_params=pltpu.CompilerParams(
            needs_layout_passes=False, vmem_limit_bytes=8 * 1024 * 1024
        ),
        out_type=jax.ShapeDtypeStruct((n_tok,), jnp.int32),
        scratch_types=[
            pltpu.VMEM((n_exp, tpw), jnp.float32),
            pltpu.VMEM((tpw,), jnp.int32),
        ],
    )(logits_t)


def kernel(logits, omega):
    logits_t = logits.T
    sample = _sc_sample(logits_t)
    multiplier = _tc_multiplier(logits_t, omega)
    return sample.reshape(-1, 1), multiplier, jnp.float32(0.0)


# speedup vs baseline: 1.0598x; 1.0031x over previous
import jax
import jax.numpy as jnp
from jax import lax
from jax.experimental import pallas as pl
from jax.experimental.pallas import tpu as pltpu
from jax.experimental.pallas import tpu_sc as plsc

_JITTER_EPS = 0.1
_TOK_BLK = 512

_SC_CORES = 2
_SC_SUBCORES = 16
_SC_LANES = 16
_SC_WORKERS = _SC_CORES * _SC_SUBCORES


def _tc_body(lg_ref, om_ref, mult_ref):
    lg = lg_ref[...]
    mx = jnp.max(lg, axis=0, keepdims=True)
    factor = jnp.maximum(jnp.abs(lg), mx)
    mask = (mx - lg) / factor > 2.0 * _JITTER_EPS
    e = jnp.where(mask, 0.0, jnp.exp(lg - mx))
    m = 1.0 / jnp.sum(e, axis=0, keepdims=True)
    mult_ref[...] = m.reshape(-1, 1) * om_ref[...][None, :]


def _tc_multiplier(logits_t, omega):
    n_exp, n_tok = logits_t.shape
    dim = omega.shape[0]
    return pl.pallas_call(
        _tc_body,
        grid=(n_tok // _TOK_BLK,),
        in_specs=[
            pl.BlockSpec((n_exp, _TOK_BLK), lambda i: (0, i)),
            pl.BlockSpec((dim,), lambda i: (0,)),
        ],
        out_specs=pl.BlockSpec((_TOK_BLK, dim), lambda i: (i, 0)),
        out_shape=jax.ShapeDtypeStruct((n_tok, dim), jnp.float32),
        compiler_params=pltpu.CompilerParams(vmem_limit_bytes=40 * 1024 * 1024),
    )(logits_t, omega)


def _sc_sample(logits_t):
    n_exp, n_tok = logits_t.shape
    tpw = n_tok // _SC_WORKERS
    n_grp = tpw // _SC_LANES

    def body(lg_hbm, out_hbm, lg_v, samp_v):
        wid = lax.axis_index("s") * _SC_CORES + lax.axis_index("c")
        base = wid * tpw
        pltpu.sync_copy(lg_hbm.at[:, pl.ds(base, tpw)], lg_v)

        def group(g, carry):
            off = g * _SC_LANES
            best_v = lg_v[0, pl.ds(off, _SC_LANES)]
            best_i = jnp.zeros((_SC_LANES,), jnp.int32)
            for j in range(1, n_exp):
                v = lg_v[j, pl.ds(off, _SC_LANES)]
                upd = v > best_v
                best_v = jnp.where(upd, v, best_v)
                best_i = jnp.where(upd, jnp.full((_SC_LANES,), j, jnp.int32), best_i)
            samp_v[pl.ds(off, _SC_LANES)] = best_i
            return carry

        lax.fori_loop(0, n_grp, group, 0)
        pltpu.sync_copy(samp_v, out_hbm.at[pl.ds(base, tpw)])

    return pl.kernel(
        body,
        mesh=plsc.VectorSubcoreMesh(core_axis_name="c", subcore_axis_name="s"),
        compiler_params=pltpu.CompilerParams(
            needs_layout_passes=False,
            vmem_limit_bytes=8 * 1024 * 1024,
            skip_device_barrier=True,
        ),
        out_type=jax.ShapeDtypeStruct((n_tok,), jnp.int32),
        scratch_types=[
            pltpu.VMEM((n_exp, tpw), jnp.float32),
            pltpu.VMEM((tpw,), jnp.int32),
        ],
    )(logits_t)


def kernel(logits, omega):
    logits_t = logits.T
    sample = _sc_sample(logits_t)
    multiplier = _tc_multiplier(logits_t, omega)
    return sample.reshape(-1, 1), multiplier, jnp.float32(0.0)
